# Initial kernel scaffold; baseline (speedup 1.0000x reference)
import jax
import jax.numpy as jnp
from jax.experimental import pallas as pl


def kernel(obs_his, era_his, pan_fut, cobs, cera, cpan):
    raise NotImplementedError("write your pallas kernel here")


# speedup vs baseline: 1.1925x; 1.1925x over previous
import functools

import jax
import jax.numpy as jnp
from jax import lax
from jax.experimental import pallas as pl
from jax.experimental.pallas import tpu as pltpu
from jax.experimental.pallas import tpu_sc as plsc

KNN = 8
NB = 128


def _topk_body(cobs_ref, cera_ref, idx_ref):
    G = cera_ref.shape[1]
    xo = cobs_ref[:, 0:1]
    yo = cobs_ref[:, 1:2]
    xg = cera_ref[0:1, :]
    yg = cera_ref[1:2, :]
    d2 = (xo - xg) ** 2 + (yo - yg) ** 2
    lane = lax.broadcasted_iota(jnp.int32, d2.shape, 1)
    work = d2
    for j in range(KNN):
        m = jnp.min(work, axis=1, keepdims=True)
        cand = jnp.where(work == m, lane, G)
        amin = jnp.min(cand, axis=1, keepdims=True)
        idx_ref[:, j : j + 1] = amin
        work = jnp.where(lane == amin, jnp.inf, work)


def _topk(cobs, cera_t):
    N = cobs.shape[0]
    G = cera_t.shape[1]
    return pl.pallas_call(
        _topk_body,
        grid=(N // NB,),
        in_specs=[
            pl.BlockSpec((NB, 2), lambda i: (i, 0)),
            pl.BlockSpec((2, G), lambda i: (0, 0)),
        ],
        out_specs=pl.BlockSpec((NB, KNN), lambda i: (i, 0)),
        out_shape=jax.ShapeDtypeStruct((N, KNN), jnp.int32),
    )(cobs, cera_t)


def _sc_gather(era2d, pan2d, cera_pad, idx_flat):
    NK = idx_flat.shape[0]
    G = cera_pad.shape[0]
    L = era2d.shape[1]
    BC = era2d.shape[0] // G
    info = plsc.get_sparse_core_info()
    NC, NS = info.num_cores, info.num_subcores
    NW = NC * NS
    assert BC == NW
    CH = 128
    NCH = NK // CH

    mesh = plsc.VectorSubcoreMesh(core_axis_name="c", subcore_axis_name="s")

    @functools.partial(
        pl.kernel,
        mesh=mesh,
        compiler_params=pltpu.CompilerParams(use_tc_tiling_on_sc=False),
        out_type=[
            jax.ShapeDtypeStruct((BC * NK, L), jnp.float32),
            jax.ShapeDtypeStruct((BC * NK, L), jnp.float32),
            jax.ShapeDtypeStruct((NK, 16), jnp.float32),
        ],
        scratch_types=[
            pltpu.VMEM((NK,), jnp.int32),
            pltpu.VMEM((CH, L), jnp.float32),
            pltpu.VMEM((CH, 16), jnp.float32),
            pltpu.SemaphoreType.DMA,
        ],
    )
    def k(era_hbm, pan_hbm, cera_hbm, idx_hbm, era_o, pan_o, cera_o,
          idx_v, buf, cbuf, sem):
        wid = lax.axis_index("s") * NC + lax.axis_index("c")
        pltpu.sync_copy(idx_hbm, idx_v)
        for t in range(NK // (CH * NW)):
            j = wid * (NK // (CH * NW)) + t
            pltpu.async_copy(
                cera_hbm.at[idx_v.at[pl.ds(j * CH, CH)]], cbuf, sem
            ).wait()
            pltpu.sync_copy(cbuf, cera_o.at[pl.ds(j * CH, CH)])
        off = lax.broadcast(wid * G, (16,))

        def addoff(i, c):
            idx_v[pl.ds(i * 16, 16)] = idx_v[pl.ds(i * 16, 16)] + off
            return c

        lax.fori_loop(0, NK // 16, addoff, 0)
        base = wid * NK
        for tab, out in ((era_hbm, era_o), (pan_hbm, pan_o)):
            def gath(j, c, tab=tab, out=out):
                pltpu.async_copy(
                    tab.at[idx_v.at[pl.ds(j * CH, CH)]], buf, sem
                ).wait()
                pltpu.sync_copy(buf, out.at[pl.ds(base + j * CH, CH)])
                return c

            lax.fori_loop(0, NCH, gath, 0)

    return k(era2d, pan2d, cera_pad, idx_flat)


def kernel(obs_his, era_his, pan_fut, cobs, cera, cpan):
    B, C, N, L = obs_his.shape
    lat, lon = era_his.shape[2], era_his.shape[3]
    G = lat * lon
    cera_flat = cera.reshape(G, 2)
    idx = _topk(cobs, cera_flat.T)
    idx_flat = idx.reshape(N * KNN)
    era2d = era_his.reshape(B * C * G, L)
    pan2d = pan_fut.reshape(B * C * G, L)
    cera_pad = jnp.pad(cera_flat, ((0, 0), (0, 14)))
    era_g, pan_g, cera_g = _sc_gather(era2d, pan2d, cera_pad, idx_flat)
    era_k = era_g.reshape(B, C, N, KNN, L)
    pan_k = pan_g.reshape(B, C, N, KNN, L)
    cera_k = cera_g[:, :2].reshape(N, KNN, 2)
    return era_k, pan_k, cera_k


# speedup vs baseline: 1.9784x; 1.6591x over previous
import functools

import jax
import jax.numpy as jnp
from jax import lax
from jax.experimental import pallas as pl
from jax.experimental.pallas import tpu as pltpu
from jax.experimental.pallas import tpu_sc as plsc

KNN = 8
NB = 128


def _topk_body(cobs_ref, cera_ref, idx_ref):
    G = cera_ref.shape[0]
    xo = cobs_ref[0:1, :]
    yo = cobs_ref[1:2, :]
    xg = cera_ref[:, 0:1]
    yg = cera_ref[:, 1:2]
    d2 = (xg - xo) ** 2 + (yg - yo) ** 2
    subl = lax.broadcasted_iota(jnp.int32, d2.shape, 0)
    inf = jnp.float32(jnp.inf)
    work = d2
    for j in range(KNN):
        m = jnp.min(work, axis=0, keepdims=True)
        cand = jnp.where(work == m, subl, G)
        amin = jnp.min(cand, axis=0, keepdims=True)
        idx_ref[:, j, :] = amin
        work = jnp.where(subl == amin, inf, work)


def _topk(cobs_t, cera_c):
    N = cobs_t.shape[1]
    G = cera_c.shape[0]
    return pl.pallas_call(
        _topk_body,
        grid=(N // NB,),
        in_specs=[
            pl.BlockSpec((2, NB), lambda i: (0, i)),
            pl.BlockSpec((G, 2), lambda i: (0, 0)),
        ],
        out_specs=pl.BlockSpec((1, KNN, NB), lambda i: (i, 0, 0)),
        out_shape=jax.ShapeDtypeStruct((N // NB, KNN, NB), jnp.int32),
    )(cobs_t, cera_c)


def _sc_gather(era3, pan3, cera2, idx_lin):
    BC = era3.shape[0]
    NK = idx_lin.shape[0]
    NCHUNK = 6
    info = plsc.get_sparse_core_info()
    NC = info.num_cores
    mesh = plsc.VectorSubcoreMesh(core_axis_name="c", subcore_axis_name="s")

    @functools.partial(
        pl.kernel,
        mesh=mesh,
        compiler_params=pltpu.CompilerParams(
            use_tc_tiling_on_sc=False, needs_layout_passes=False),
        out_type=[
            jax.ShapeDtypeStruct((BC, KNN, NCHUNK, 8192), jnp.float32),
            jax.ShapeDtypeStruct((BC, KNN, NCHUNK, 8192), jnp.float32),
            jax.ShapeDtypeStruct((KNN, 2, 1024), jnp.float32),
        ],
        scratch_types=[
            pltpu.VMEM((NK,), jnp.int32),
            pltpu.VMEM((64, 1024), jnp.float32),
            pltpu.VMEM((8192,), jnp.float32),
            pltpu.VMEM((64, 256), jnp.float32),
            pltpu.VMEM((2, 1024), jnp.float32),
        ],
    )
    def k(era_h, pan_h, cera_h, idx_h, era_o, pan_o, cera_o,
          idx_v, tab_v, stage_v, cer_v, stc_v):
        wid = lax.axis_index("s") * NC + lax.axis_index("c")
        pltpu.sync_copy(idx_h, idx_v)

        @pl.when(wid < KNN)
        def _cera():
            pltpu.sync_copy(cera_h, cer_v)
            for coord in range(2):
                def cnj(t, c, coord=coord):
                    nt = t // 8
                    j = t - nt * 8
                    v = idx_v[pl.ds(nt * 1024 + wid * 128 + j * 16, 16)]
                    la = lax.shift_right_logical(v, 7)
                    lo = v & 127
                    g = plsc.load_gather(cer_v, [la, lo + (coord * 128)])
                    stc_v[coord, pl.ds(nt * 128 + j * 16, 16)] = g
                    return c
                lax.fori_loop(0, 64, cnj, 0)
            pltpu.sync_copy(stc_v, cera_o.at[wid])

        for tab_h, out_h in ((era_h, era_o), (pan_h, pan_o)):
            def chunk(lt, c, tab_h=tab_h, out_h=out_h):
                pltpu.sync_copy(tab_h.at[wid, :, pl.ds(lt * 1024, 1024)],
                                tab_v)
                def perk(kk, c2):
                    def nj(t, c3, kk=kk):
                        nt = t // 8
                        j = t - nt * 8
                        v = idx_v[pl.ds(nt * 1024 + kk * 128 + j * 16, 16)]
                        la = lax.shift_right_logical(v, 7)
                        lo = v & 127
                        for l8 in range(8):
                            g = plsc.load_gather(tab_v, [la, lo + (l8 * 128)])
                            stage_v[pl.ds((nt * 8 + l8) * 128 + j * 16, 16)] = g
                        return c3
                    lax.fori_loop(0, 64, nj, 0)
                    pltpu.sync_copy(stage_v, out_h.at[wid, kk, lt])
                    return c2
                lax.fori_loop(0, KNN, perk, 0)
                return c
            lax.fori_loop(0, NCHUNK, chunk, 0)

    return k(era3, pan3, cera2, idx_lin)


def kernel(obs_his, era_his, pan_fut, cobs, cera, cpan):
    B, C, N, L = obs_his.shape
    lat, lon = era_his.shape[2], era_his.shape[3]
    G = lat * lon
    era3 = era_his.transpose(0, 1, 2, 4, 3).reshape(B * C, lat, L * lon)
    pan3 = pan_fut.transpose(0, 1, 2, 4, 3).reshape(B * C, lat, L * lon)
    cera2 = cera.transpose(0, 2, 1).reshape(lat, 2 * lon)
    idx_t3 = _topk(cobs.T, cera.reshape(G, 2))
    idx_lin = idx_t3.reshape(N * KNN)
    era7, pan7, cer2 = _sc_gather(era3, pan3, cera2, idx_lin)
    def detile(x):
        x = x.reshape(B, C, KNN, 6, 8, 8, 128)
        x = x.transpose(0, 1, 4, 6, 2, 3, 5)
        return x.reshape(B, C, N, KNN, L)
    era_k = detile(era7)
    pan_k = detile(pan7)
    cera_k = cer2.transpose(2, 0, 1)
    return era_k, pan_k, cera_k


# speedup vs baseline: 2.1094x; 1.0662x over previous
import functools

import jax
import jax.numpy as jnp
from jax import lax
from jax.experimental import pallas as pl
from jax.experimental.pallas import tpu as pltpu
from jax.experimental.pallas import tpu_sc as plsc

KNN = 8
NB = 128


def _topk_body(cobs_ref, cera_ref, idx_ref):
    G = cera_ref.shape[0]
    xo = cobs_ref[0:1, :]
    yo = cobs_ref[1:2, :]
    xg = cera_ref[:, 0:1]
    yg = cera_ref[:, 1:2]
    d2 = (xg - xo) ** 2 + (yg - yo) ** 2
    subl = lax.broadcasted_iota(jnp.int32, d2.shape, 0)
    inf = jnp.float32(jnp.inf)
    work = d2
    for j in range(KNN):
        m = jnp.min(work, axis=0, keepdims=True)
        cand = jnp.where(work == m, subl, G)
        amin = jnp.min(cand, axis=0, keepdims=True)
        idx_ref[:, j, :] = amin
        work = jnp.where(subl == amin, inf, work)


def _topk(cobs_t, cera_c):
    N = cobs_t.shape[1]
    G = cera_c.shape[0]
    return pl.pallas_call(
        _topk_body,
        grid=(N // NB,),
        in_specs=[
            pl.BlockSpec((2, NB), lambda i: (0, i)),
            pl.BlockSpec((G, 2), lambda i: (0, 0)),
        ],
        out_specs=pl.BlockSpec((1, KNN, NB), lambda i: (i, 0, 0)),
        out_shape=jax.ShapeDtypeStruct((N // NB, KNN, NB), jnp.int32),
    )(cobs_t, cera_c)


def _sc_gather(era3, pan3, cera2, idx_lin):
    BC = era3.shape[0]
    NK = idx_lin.shape[0]
    NCHUNK = 6
    info = plsc.get_sparse_core_info()
    NC = info.num_cores
    mesh = plsc.VectorSubcoreMesh(core_axis_name="c", subcore_axis_name="s")

    @functools.partial(
        pl.kernel,
        mesh=mesh,
        compiler_params=pltpu.CompilerParams(
            use_tc_tiling_on_sc=False, needs_layout_passes=False),
        out_type=[
            jax.ShapeDtypeStruct((BC, KNN, NCHUNK, 8192), jnp.float32),
            jax.ShapeDtypeStruct((BC, KNN, NCHUNK, 8192), jnp.float32),
            jax.ShapeDtypeStruct((KNN, 2, 1024), jnp.float32),
        ],
        scratch_types=[
            pltpu.VMEM((NK,), jnp.int32),
            pltpu.VMEM((64, 1024), jnp.float32),
            pltpu.VMEM((8192,), jnp.float32),
            pltpu.VMEM((8192,), jnp.float32),
            pltpu.VMEM((64, 256), jnp.float32),
            pltpu.VMEM((2, 1024), jnp.float32),
            pltpu.SemaphoreType.DMA,
            pltpu.SemaphoreType.DMA,
        ],
    )
    def k(era_h, pan_h, cera_h, idx_h, era_o, pan_o, cera_o,
          idx_v, tab_v, st0_v, st1_v, cer_v, stc_v, ss0, ss1):
        wid = lax.axis_index("s") * NC + lax.axis_index("c")
        pltpu.sync_copy(idx_h, idx_v)

        @pl.when(wid < KNN)
        def _cera():
            pltpu.sync_copy(cera_h, cer_v)
            for coord in range(2):
                def cnj(t, c, coord=coord):
                    nt = t // 8
                    j = t - nt * 8
                    v = idx_v[pl.ds(nt * 1024 + wid * 128 + j * 16, 16)]
                    la = lax.shift_right_logical(v, 7)
                    lo = v & 127
                    g = plsc.load_gather(cer_v, [la, lo + (coord * 128)])
                    stc_v[coord, pl.ds(nt * 128 + j * 16, 16)] = g
                    return c
                lax.fori_loop(0, 64, cnj, 0)
            pltpu.sync_copy(stc_v, cera_o.at[wid])

        stages = (st0_v, st1_v)
        ssems = (ss0, ss1)

        def drain(sb):
            pltpu.make_async_copy(stages[sb], era_o.at[wid, 0, 0],
                                  ssems[sb]).wait()

        for ti, (tab_h, out_h) in enumerate(((era_h, era_o),
                                             (pan_h, pan_o))):
            def chunk(lt, c, tab_h=tab_h, out_h=out_h, ti=ti):
                pltpu.sync_copy(tab_h.at[wid, :, pl.ds(lt * 1024, 1024)],
                                tab_v)
                for kk in range(KNN):
                    sb = kk % 2
                    if kk >= 2:
                        drain(sb)
                    else:
                        cond = (lt > 0) if ti == 0 else (lt >= 0)

                        @pl.when(cond)
                        def _w(sb=sb):
                            drain(sb)
                    stage_v = stages[sb]

                    def nj(t, c3, kk=kk, stage_v=stage_v):
                        nt = t // 8
                        j = t - nt * 8
                        v = idx_v[pl.ds(nt * 1024 + kk * 128 + j * 16, 16)]
                        la = lax.shift_right_logical(v, 7)
                        lo = v & 127
                        for l8 in range(8):
                            g = plsc.load_gather(tab_v, [la, lo + (l8 * 128)])
                            stage_v[pl.ds((nt * 8 + l8) * 128 + j * 16, 16)] = g
                        return c3
                    lax.fori_loop(0, 64, nj, 0)
                    pltpu.async_copy(stage_v, out_h.at[wid, kk, lt],
                                     ssems[sb])
                return c
            lax.fori_loop(0, NCHUNK, chunk, 0)
        drain(0)
        drain(1)

    return k(era3, pan3, cera2, idx_lin)


def kernel(obs_his, era_his, pan_fut, cobs, cera, cpan):
    B, C, N, L = obs_his.shape
    lat, lon = era_his.shape[2], era_his.shape[3]
    G = lat * lon
    era3 = era_his.transpose(0, 1, 2, 4, 3).reshape(B * C, lat, L * lon)
    pan3 = pan_fut.transpose(0, 1, 2, 4, 3).reshape(B * C, lat, L * lon)
    cera2 = cera.transpose(0, 2, 1).reshape(lat, 2 * lon)
    idx_t3 = _topk(cobs.T, cera.reshape(G, 2))
    idx_lin = idx_t3.reshape(N * KNN)
    era7, pan7, cer2 = _sc_gather(era3, pan3, cera2, idx_lin)
    def detile(x):
        x = x.reshape(B, C, KNN, 6, 8, 8, 128)
        x = x.transpose(0, 1, 4, 6, 2, 3, 5)
        return x.reshape(B, C, N, KNN, L)
    era_k = detile(era7)
    pan_k = detile(pan7)
    cera_k = cer2.transpose(2, 0, 1)
    return era_k, pan_k, cera_k


# speedup vs baseline: 3.8112x; 1.8067x over previous
import functools

import jax
import jax.numpy as jnp
from jax import lax
from jax.experimental import pallas as pl
from jax.experimental.pallas import tpu as pltpu
from jax.experimental.pallas import tpu_sc as plsc

KNN = 8
NB = 128


def _topk_body(cobs_ref, cera_ref, idx_ref):
    G = cera_ref.shape[0]
    xo = cobs_ref[0:1, :]
    yo = cobs_ref[1:2, :]
    xg = cera_ref[:, 0:1]
    yg = cera_ref[:, 1:2]
    d2 = (xg - xo) ** 2 + (yg - yo) ** 2
    subl = lax.broadcasted_iota(jnp.int32, d2.shape, 0)
    inf = jnp.float32(jnp.inf)
    work = d2
    for j in range(KNN):
        m = jnp.min(work, axis=0, keepdims=True)
        cand = jnp.where(work == m, subl, G)
        amin = jnp.min(cand, axis=0, keepdims=True)
        idx_ref[:, j, :] = amin
        work = jnp.where(subl == amin, inf, work)


def _topk(cobs_t, cera_c):
    N = cobs_t.shape[1]
    G = cera_c.shape[0]
    return pl.pallas_call(
        _topk_body,
        grid=(N // NB,),
        in_specs=[
            pl.BlockSpec((2, NB), lambda i: (0, i)),
            pl.BlockSpec((G, 2), lambda i: (0, 0)),
        ],
        out_specs=pl.BlockSpec((1, KNN, NB), lambda i: (i, 0, 0)),
        out_shape=jax.ShapeDtypeStruct((N // NB, KNN, NB), jnp.int32),
    )(cobs_t, cera_c)


def _sc_gather(era3, pan3, cera2, idx_lin):
    BC = era3.shape[0]
    NK = idx_lin.shape[0]
    NCHUNK = 6
    info = plsc.get_sparse_core_info()
    NC = info.num_cores
    mesh = plsc.VectorSubcoreMesh(core_axis_name="c", subcore_axis_name="s")

    @functools.partial(
        pl.kernel,
        mesh=mesh,
        compiler_params=pltpu.CompilerParams(
            use_tc_tiling_on_sc=False, needs_layout_passes=False),
        out_type=[
            jax.ShapeDtypeStruct((BC, KNN, NCHUNK, 8192), jnp.float32),
            jax.ShapeDtypeStruct((BC, KNN, NCHUNK, 8192), jnp.float32),
            jax.ShapeDtypeStruct((KNN, 2, 1024), jnp.float32),
        ],
        scratch_types=[
            pltpu.VMEM((NK,), jnp.int32),
            pltpu.VMEM((64, 1024), jnp.float32),
            pltpu.VMEM((8192,), jnp.float32),
            pltpu.VMEM((8192,), jnp.float32),
            pltpu.VMEM((64, 256), jnp.float32),
            pltpu.VMEM((2, 1024), jnp.float32),
            pltpu.SemaphoreType.DMA,
            pltpu.SemaphoreType.DMA,
        ],
    )
    def k(era_h, pan_h, cera_h, idx_h, era_o, pan_o, cera_o,
          idx_v, tab_v, st0_v, st1_v, cer_v, stc_v, ss0, ss1):
        wid = lax.axis_index("s") * NC + lax.axis_index("c")
        pltpu.sync_copy(idx_h, idx_v)

        @pl.when(wid < KNN)
        def _cera():
            pltpu.sync_copy(cera_h, cer_v)
            for coord in range(2):
                @plsc.parallel_loop(0, 64, unroll=2)
                def cnj(t, coord=coord):
                    nt = t // 8
                    j = t - nt * 8
                    v = idx_v[pl.ds(nt * 1024 + wid * 128 + j * 16, 16)]
                    la = lax.shift_right_logical(v, 7)
                    lo = v & 127
                    g = plsc.load_gather(cer_v, [la, lo + (coord * 128)])
                    stc_v[coord, pl.ds(nt * 128 + j * 16, 16)] = g
            pltpu.sync_copy(stc_v, cera_o.at[wid])

        stages = (st0_v, st1_v)
        ssems = (ss0, ss1)

        def drain(sb):
            pltpu.make_async_copy(stages[sb], era_o.at[wid, 0, 0],
                                  ssems[sb]).wait()

        for ti, (tab_h, out_h) in enumerate(((era_h, era_o),
                                             (pan_h, pan_o))):
            def chunk(lt, c, tab_h=tab_h, out_h=out_h, ti=ti):
                pltpu.sync_copy(tab_h.at[wid, :, pl.ds(lt * 1024, 1024)],
                                tab_v)
                for kk in range(KNN):
                    sb = kk % 2
                    if kk >= 2:
                        drain(sb)
                    else:
                        cond = (lt > 0) if ti == 0 else (lt >= 0)

                        @pl.when(cond)
                        def _w(sb=sb):
                            drain(sb)
                    stage_v = stages[sb]

                    @plsc.parallel_loop(0, 64, unroll=2)
                    def nj(t, kk=kk, stage_v=stage_v):
                        nt = t // 8
                        j = t - nt * 8
                        v = idx_v[pl.ds(nt * 1024 + kk * 128 + j * 16, 16)]
                        la = lax.shift_right_logical(v, 7)
                        lo = v & 127
                        for l8 in range(8):
                            g = plsc.load_gather(tab_v, [la, lo + (l8 * 128)])
                            stage_v[pl.ds((nt * 8 + l8) * 128 + j * 16, 16)] = g
                    pltpu.async_copy(stage_v, out_h.at[wid, kk, lt],
                                     ssems[sb])
                return c
            lax.fori_loop(0, NCHUNK, chunk, 0)
        drain(0)
        drain(1)

    return k(era3, pan3, cera2, idx_lin)


def kernel(obs_his, era_his, pan_fut, cobs, cera, cpan):
    B, C, N, L = obs_his.shape
    lat, lon = era_his.shape[2], era_his.shape[3]
    G = lat * lon
    era3 = era_his.transpose(0, 1, 2, 4, 3).reshape(B * C, lat, L * lon)
    pan3 = pan_fut.transpose(0, 1, 2, 4, 3).reshape(B * C, lat, L * lon)
    cera2 = cera.transpose(0, 2, 1).reshape(lat, 2 * lon)
    idx_t3 = _topk(cobs.T, cera.reshape(G, 2))
    idx_lin = idx_t3.reshape(N * KNN)
    era7, pan7, cer2 = _sc_gather(era3, pan3, cera2, idx_lin)
    def detile(x):
        x = x.reshape(B, C, KNN, 6, 8, 8, 128)
        x = x.transpose(0, 1, 4, 6, 2, 3, 5)
        return x.reshape(B, C, N, KNN, L)
    era_k = detile(era7)
    pan_k = detile(pan7)
    cera_k = cer2.transpose(2, 0, 1)
    return era_k, pan_k, cera_k


# speedup vs baseline: 3.8163x; 1.0013x over previous
import functools

import jax
import jax.numpy as jnp
from jax import lax
from jax.experimental import pallas as pl
from jax.experimental.pallas import tpu as pltpu
from jax.experimental.pallas import tpu_sc as plsc

KNN = 8
NB = 128


def _topk_body(cobs_ref, cera_ref, idx_ref):
    G = cera_ref.shape[0]
    xo = cobs_ref[0:1, :]
    yo = cobs_ref[1:2, :]
    xg = cera_ref[:, 0:1]
    yg = cera_ref[:, 1:2]
    d2 = (xg - xo) ** 2 + (yg - yo) ** 2
    subl = lax.broadcasted_iota(jnp.int32, d2.shape, 0)
    inf = jnp.float32(jnp.inf)
    work = d2
    for j in range(KNN):
        m = jnp.min(work, axis=0, keepdims=True)
        cand = jnp.where(work == m, subl, G)
        amin = jnp.min(cand, axis=0, keepdims=True)
        idx_ref[:, j, :] = amin
        work = jnp.where(subl == amin, inf, work)


def _topk(cobs_t, cera_c):
    N = cobs_t.shape[1]
    G = cera_c.shape[0]
    return pl.pallas_call(
        _topk_body,
        grid=(N // NB,),
        in_specs=[
            pl.BlockSpec((2, NB), lambda i: (0, i)),
            pl.BlockSpec((G, 2), lambda i: (0, 0)),
        ],
        out_specs=pl.BlockSpec((1, KNN, NB), lambda i: (i, 0, 0)),
        out_shape=jax.ShapeDtypeStruct((N // NB, KNN, NB), jnp.int32),
    )(cobs_t, cera_c)


def _sc_gather(era3, pan3, cera2, idx_lin):
    BC = era3.shape[0]
    NK = idx_lin.shape[0]
    NCHUNK = 6
    info = plsc.get_sparse_core_info()
    NC = info.num_cores
    mesh = plsc.VectorSubcoreMesh(core_axis_name="c", subcore_axis_name="s")

    @functools.partial(
        pl.kernel,
        mesh=mesh,
        compiler_params=pltpu.CompilerParams(
            use_tc_tiling_on_sc=False, needs_layout_passes=False),
        out_type=[
            jax.ShapeDtypeStruct((BC, KNN, NCHUNK, 8192), jnp.float32),
            jax.ShapeDtypeStruct((BC, KNN, NCHUNK, 8192), jnp.float32),
            jax.ShapeDtypeStruct((KNN, 2, 1024), jnp.float32),
        ],
        scratch_types=[
            pltpu.VMEM((NK,), jnp.int32),
            pltpu.VMEM((64, 1024), jnp.float32),
            pltpu.VMEM((8192,), jnp.float32),
            pltpu.VMEM((8192,), jnp.float32),
            pltpu.VMEM((64, 256), jnp.float32),
            pltpu.VMEM((2, 1024), jnp.float32),
            pltpu.SemaphoreType.DMA,
            pltpu.SemaphoreType.DMA,
        ],
    )
    def k(era_h, pan_h, cera_h, idx_h, era_o, pan_o, cera_o,
          idx_v, tab_v, st0_v, st1_v, cer_v, stc_v, ss0, ss1):
        wid = lax.axis_index("s") * NC + lax.axis_index("c")
        pltpu.sync_copy(idx_h, idx_v)

        @pl.when(wid < KNN)
        def _cera():
            pltpu.sync_copy(cera_h, cer_v)
            for coord in range(2):
                @plsc.parallel_loop(0, 64, unroll=2)
                def cnj(t, coord=coord):
                    nt = t // 8
                    j = t - nt * 8
                    v = idx_v[pl.ds(nt * 1024 + wid * 128 + j * 16, 16)]
                    la = lax.shift_right_logical(v, 7)
                    lo = v & 127
                    g = plsc.load_gather(cer_v, [la, lo + (coord * 128)])
                    stc_v[coord, pl.ds(nt * 128 + j * 16, 16)] = g
            pltpu.sync_copy(stc_v, cera_o.at[wid])

        stages = (st0_v, st1_v)
        ssems = (ss0, ss1)

        def drain(sb):
            pltpu.make_async_copy(stages[sb], era_o.at[wid, 0, 0],
                                  ssems[sb]).wait()

        for ti, (tab_h, out_h) in enumerate(((era_h, era_o),
                                             (pan_h, pan_o))):
            def chunk(lt, c, tab_h=tab_h, out_h=out_h, ti=ti):
                pltpu.sync_copy(tab_h.at[wid, :, pl.ds(lt * 1024, 1024)],
                                tab_v)
                for kk in range(KNN):
                    sb = kk % 2
                    if kk >= 2:
                        drain(sb)
                    else:
                        cond = (lt > 0) if ti == 0 else (lt >= 0)

                        @pl.when(cond)
                        def _w(sb=sb):
                            drain(sb)
                    stage_v = stages[sb]

                    @plsc.parallel_loop(0, 64, unroll=4)
                    def nj(t, kk=kk, stage_v=stage_v):
                        nt = t // 8
                        j = t - nt * 8
                        v = idx_v[pl.ds(nt * 1024 + kk * 128 + j * 16, 16)]
                        la = lax.shift_right_logical(v, 7)
                        lo = v & 127
                        for l8 in range(8):
                            g = plsc.load_gather(tab_v, [la, lo + (l8 * 128)])
                            stage_v[pl.ds((nt * 8 + l8) * 128 + j * 16, 16)] = g
                    pltpu.async_copy(stage_v, out_h.at[wid, kk, lt],
                                     ssems[sb])
                return c
            lax.fori_loop(0, NCHUNK, chunk, 0)
        drain(0)
        drain(1)

    return k(era3, pan3, cera2, idx_lin)


def kernel(obs_his, era_his, pan_fut, cobs, cera, cpan):
    B, C, N, L = obs_his.shape
    lat, lon = era_his.shape[2], era_his.shape[3]
    G = lat * lon
    era3 = era_his.transpose(0, 1, 2, 4, 3).reshape(B * C, lat, L * lon)
    pan3 = pan_fut.transpose(0, 1, 2, 4, 3).reshape(B * C, lat, L * lon)
    cera2 = cera.transpose(0, 2, 1).reshape(lat, 2 * lon)
    idx_t3 = _topk(cobs.T, cera.reshape(G, 2))
    idx_lin = idx_t3.reshape(N * KNN)
    era7, pan7, cer2 = _sc_gather(era3, pan3, cera2, idx_lin)
    def detile(x):
        x = x.reshape(B, C, KNN, 6, 8, 8, 128)
        x = x.transpose(0, 1, 4, 6, 2, 3, 5)
        return x.reshape(B, C, N, KNN, L)
    era_k = detile(era7)
    pan_k = detile(pan7)
    cera_k = cer2.transpose(2, 0, 1)
    return era_k, pan_k, cera_k


# speedup vs baseline: 4.8107x; 1.2606x over previous
import functools

import jax
import jax.numpy as jnp
from jax import lax
from jax.experimental import pallas as pl
from jax.experimental.pallas import tpu as pltpu
from jax.experimental.pallas import tpu_sc as plsc

KNN = 8
NB = 128


def _topk_body(cobs_ref, cera_ref, idx_ref):
    G = cera_ref.shape[0]
    xo = cobs_ref[0:1, :]
    yo = cobs_ref[1:2, :]
    xg = cera_ref[:, 0:1]
    yg = cera_ref[:, 1:2]
    d2 = (xg - xo) ** 2 + (yg - yo) ** 2
    subl = lax.broadcasted_iota(jnp.int32, d2.shape, 0)
    inf = jnp.float32(jnp.inf)
    work = d2
    for j in range(KNN):
        amin = jnp.argmin(work, axis=0).astype(jnp.int32)[None, :]
        idx_ref[:, j, :] = amin
        work = jnp.where(subl == amin, inf, work)


def _topk(cobs_t, cera_c):
    N = cobs_t.shape[1]
    G = cera_c.shape[0]
    return pl.pallas_call(
        _topk_body,
        grid=(N // NB,),
        in_specs=[
            pl.BlockSpec((2, NB), lambda i: (0, i)),
            pl.BlockSpec((G, 2), lambda i: (0, 0)),
        ],
        out_specs=pl.BlockSpec((1, KNN, NB), lambda i: (i, 0, 0)),
        out_shape=jax.ShapeDtypeStruct((N // NB, KNN, NB), jnp.int32),
    )(cobs_t, cera_c)


def _sc_gather(era3, pan3, cera2, idx_lin):
    BC = era3.shape[0]
    NK = idx_lin.shape[0]
    NCHUNK = 6
    info = plsc.get_sparse_core_info()
    NC = info.num_cores
    mesh = plsc.VectorSubcoreMesh(core_axis_name="c", subcore_axis_name="s")

    @functools.partial(
        pl.kernel,
        mesh=mesh,
        compiler_params=pltpu.CompilerParams(
            use_tc_tiling_on_sc=False, needs_layout_passes=False),
        out_type=[
            jax.ShapeDtypeStruct((BC, KNN, NCHUNK, 8192), jnp.float32),
            jax.ShapeDtypeStruct((BC, KNN, NCHUNK, 8192), jnp.float32),
            jax.ShapeDtypeStruct((KNN, 2, 1024), jnp.float32),
        ],
        scratch_types=[
            pltpu.VMEM((NK,), jnp.int32),
            pltpu.VMEM((64, 1024), jnp.float32),
            pltpu.VMEM((8192,), jnp.float32),
            pltpu.VMEM((8192,), jnp.float32),
            pltpu.VMEM((64, 256), jnp.float32),
            pltpu.VMEM((2, 1024), jnp.float32),
            pltpu.SemaphoreType.DMA,
            pltpu.SemaphoreType.DMA,
        ],
    )
    def k(era_h, pan_h, cera_h, idx_h, era_o, pan_o, cera_o,
          idx_v, tab_v, st0_v, st1_v, cer_v, stc_v, ss0, ss1):
        wid = lax.axis_index("s") * NC + lax.axis_index("c")
        pltpu.sync_copy(idx_h, idx_v)

        @pl.when(wid < KNN)
        def _cera():
            pltpu.sync_copy(cera_h, cer_v)
            for coord in range(2):
                @plsc.parallel_loop(0, 64, unroll=2)
                def cnj(t, coord=coord):
                    nt = t // 8
                    j = t - nt * 8
                    v = idx_v[pl.ds(nt * 1024 + wid * 128 + j * 16, 16)]
                    la = lax.shift_right_logical(v, 7)
                    lo = v & 127
                    g = plsc.load_gather(cer_v, [la, lo + (coord * 128)])
                    stc_v[coord, pl.ds(nt * 128 + j * 16, 16)] = g
            pltpu.sync_copy(stc_v, cera_o.at[wid])

        stages = (st0_v, st1_v)
        ssems = (ss0, ss1)

        def drain(sb):
            pltpu.make_async_copy(stages[sb], era_o.at[wid, 0, 0],
                                  ssems[sb]).wait()

        for ti, (tab_h, out_h) in enumerate(((era_h, era_o),
                                             (pan_h, pan_o))):
            def chunk(lt, c, tab_h=tab_h, out_h=out_h, ti=ti):
                pltpu.sync_copy(tab_h.at[wid, :, pl.ds(lt * 1024, 1024)],
                                tab_v)
                for kk in range(KNN):
                    sb = kk % 2
                    if kk >= 2:
                        drain(sb)
                    else:
                        cond = (lt > 0) if ti == 0 else (lt >= 0)

                        @pl.when(cond)
                        def _w(sb=sb):
                            drain(sb)
                    stage_v = stages[sb]

                    @plsc.parallel_loop(0, 64, unroll=4)
                    def nj(t, kk=kk, stage_v=stage_v):
                        nt = t // 8
                        j = t - nt * 8
                        v = idx_v[pl.ds(nt * 1024 + kk * 128 + j * 16, 16)]
                        la = lax.shift_right_logical(v, 7)
                        lo = v & 127
                        for l8 in range(8):
                            g = plsc.load_gather(tab_v, [la, lo + (l8 * 128)])
                            stage_v[pl.ds((nt * 8 + l8) * 128 + j * 16, 16)] = g
                    pltpu.async_copy(stage_v, out_h.at[wid, kk, lt],
                                     ssems[sb])
                return c
            lax.fori_loop(0, NCHUNK, chunk, 0)
        drain(0)
        drain(1)

    return k(era3, pan3, cera2, idx_lin)


def kernel(obs_his, era_his, pan_fut, cobs, cera, cpan):
    B, C, N, L = obs_his.shape
    lat, lon = era_his.shape[2], era_his.shape[3]
    G = lat * lon
    era3 = era_his.transpose(0, 1, 2, 4, 3).reshape(B * C, lat, L * lon)
    pan3 = pan_fut.transpose(0, 1, 2, 4, 3).reshape(B * C, lat, L * lon)
    cera2 = cera.transpose(0, 2, 1).reshape(lat, 2 * lon)
    idx_t3 = _topk(cobs.T, cera.reshape(G, 2))
    idx_lin = idx_t3.reshape(N * KNN)
    era7, pan7, cer2 = _sc_gather(era3, pan3, cera2, idx_lin)
    def detile(x):
        x = x.reshape(B, C, KNN, 6, 8, 8, 128)
        x = x.transpose(0, 1, 4, 6, 2, 3, 5)
        return x.reshape(B, C, N, KNN, L)
    era_k = detile(era7)
    pan_k = detile(pan7)
    cera_k = cer2.transpose(2, 0, 1)
    return era_k, pan_k, cera_k


# speedup vs baseline: 5.2145x; 1.0839x over previous
import functools

import jax
import jax.numpy as jnp
from jax import lax
from jax.experimental import pallas as pl
from jax.experimental.pallas import tpu as pltpu
from jax.experimental.pallas import tpu_sc as plsc

KNN = 8
NB = 128


def _topk_body(cobs_ref, cera_ref, idx_ref):
    G = cera_ref.shape[0]
    xo = cobs_ref[0:1, :]
    yo = cobs_ref[1:2, :]
    xg = cera_ref[:, 0:1]
    yg = cera_ref[:, 1:2]
    d2 = (xg - xo) ** 2 + (yg - yo) ** 2
    subl = lax.broadcasted_iota(jnp.int32, d2.shape, 0)
    inf = jnp.float32(jnp.inf)
    work = d2
    for j in range(KNN):
        amin = jnp.argmin(work, axis=0).astype(jnp.int32)[None, :]
        idx_ref[:, j, :] = amin
        work = jnp.where(subl == amin, inf, work)


def _topk(cobs_t, cera_c):
    N = cobs_t.shape[1]
    G = cera_c.shape[0]
    return pl.pallas_call(
        _topk_body,
        grid=(N // NB,),
        in_specs=[
            pl.BlockSpec((2, NB), lambda i: (0, i)),
            pl.BlockSpec((G, 2), lambda i: (0, 0)),
        ],
        out_specs=pl.BlockSpec((1, KNN, NB), lambda i: (i, 0, 0)),
        out_shape=jax.ShapeDtypeStruct((N // NB, KNN, NB), jnp.int32),
    )(cobs_t, cera_c)


def _sc_gather(era3, pan3, cera2, idx_lin):
    BC = era3.shape[0]
    NK = idx_lin.shape[0]
    NCHUNK = 6
    info = plsc.get_sparse_core_info()
    NC = info.num_cores
    mesh = plsc.VectorSubcoreMesh(core_axis_name="c", subcore_axis_name="s")

    @functools.partial(
        pl.kernel,
        mesh=mesh,
        compiler_params=pltpu.CompilerParams(
            use_tc_tiling_on_sc=False, needs_layout_passes=False),
        out_type=[
            jax.ShapeDtypeStruct((BC, KNN, NCHUNK, 8, 8, 128), jnp.float32),
            jax.ShapeDtypeStruct((BC, KNN, NCHUNK, 8, 8, 128), jnp.float32),
            jax.ShapeDtypeStruct((KNN, 2, 1024), jnp.float32),
        ],
        scratch_types=[
            pltpu.VMEM((NK,), jnp.int32),
            pltpu.VMEM((64, 512), jnp.float32),
            pltpu.VMEM((64, 512), jnp.float32),
            pltpu.VMEM((8, 4, 128), jnp.float32),
            pltpu.VMEM((8, 4, 128), jnp.float32),
            pltpu.VMEM((64, 256), jnp.float32),
            pltpu.VMEM((2, 1024), jnp.float32),
            pltpu.SemaphoreType.DMA,
            pltpu.SemaphoreType.DMA,
            pltpu.SemaphoreType.DMA,
            pltpu.SemaphoreType.DMA,
        ],
    )
    def k(era_h, pan_h, cera_h, idx_h, era_o, pan_o, cera_o,
          idx_v, tb0_v, tb1_v, st0_v, st1_v, cer_v, stc_v,
          ss0, ss1, ts0, ts1):
        wid = lax.axis_index("s") * NC + lax.axis_index("c")
        pltpu.sync_copy(idx_h, idx_v)

        @pl.when(wid < KNN)
        def _cera():
            pltpu.sync_copy(cera_h, cer_v)
            for coord in range(2):
                @plsc.parallel_loop(0, 64, unroll=2)
                def cnj(t, coord=coord):
                    nt = t // 8
                    j = t - nt * 8
                    v = idx_v[pl.ds(nt * 1024 + wid * 128 + j * 16, 16)]
                    la = lax.shift_right_logical(v, 7)
                    lo = v & 127
                    g = plsc.load_gather(cer_v, [la, lo + (coord * 128)])
                    stc_v[coord, pl.ds(nt * 128 + j * 16, 16)] = g
            pltpu.sync_copy(stc_v, cera_o.at[wid])

        stages = (st0_v, st1_v)
        ssems = (ss0, ss1)
        tabs = (tb0_v, tb1_v)
        tsems = (ts0, ts1)

        def sdrain(sb):
            pltpu.make_async_copy(stages[sb],
                                  era_o.at[wid, 0, 0, :, pl.ds(0, 4), :],
                                  ssems[sb]).wait()

        def tdrain(tb, tab_h):
            pltpu.make_async_copy(tab_h.at[wid, :, pl.ds(0, 512)],
                                  tabs[tb], tsems[tb]).wait()

        def gather_block(tab_v, out_h, m, off4):
            for kk in range(KNN):
                sb = kk % 2
                sdrain(sb)
                stage_v = stages[sb]

                @plsc.parallel_loop(0, 64, unroll=4)
                def nj(t, kk=kk, stage_v=stage_v, tab_v=tab_v):
                    nt = t // 8
                    j = t - nt * 8
                    v = idx_v[pl.ds(nt * 1024 + kk * 128 + j * 16, 16)]
                    la = lax.shift_right_logical(v, 7)
                    lo = v & 127
                    for l4 in range(4):
                        g = plsc.load_gather(tab_v, [la, lo + (l4 * 128)])
                        stage_v[nt, l4, pl.ds(j * 16, 16)] = g
                pltpu.async_copy(
                    stage_v, out_h.at[wid, kk, m, :, pl.ds(off4, 4), :],
                    ssems[sb])

        for sb in range(2):
            pltpu.async_copy(stages[sb],
                             era_o.at[wid, sb, 0, :, pl.ds(0, 4), :],
                             ssems[sb])

        for ti, (tab_h, out_h) in enumerate(((era_h, era_o),
                                             (pan_h, pan_o))):
            pltpu.async_copy(tab_h.at[wid, :, pl.ds(0, 512)], tabs[0],
                             tsems[0])

            def dchunk(m, c, tab_h=tab_h, out_h=out_h):
                tdrain(0, tab_h)
                pltpu.async_copy(
                    tab_h.at[wid, :, pl.ds((2 * m + 1) * 512, 512)],
                    tabs[1], tsems[1])
                gather_block(tabs[0], out_h, m, 0)
                tdrain(1, tab_h)

                @pl.when(m < NCHUNK - 1)
                def _pf():
                    pltpu.async_copy(
                        tab_h.at[wid, :, pl.ds((2 * m + 2) * 512, 512)],
                        tabs[0], tsems[0])
                gather_block(tabs[1], out_h, m, 4)
                return c
            lax.fori_loop(0, NCHUNK, dchunk, 0)
        sdrain(0)
        sdrain(1)

    return k(era3, pan3, cera2, idx_lin)


def kernel(obs_his, era_his, pan_fut, cobs, cera, cpan):
    B, C, N, L = obs_his.shape
    lat, lon = era_his.shape[2], era_his.shape[3]
    G = lat * lon
    era3 = era_his.transpose(0, 1, 2, 4, 3).reshape(B * C, lat, L * lon)
    pan3 = pan_fut.transpose(0, 1, 2, 4, 3).reshape(B * C, lat, L * lon)
    cera2 = cera.transpose(0, 2, 1).reshape(lat, 2 * lon)
    idx_t3 = _topk(cobs.T, cera.reshape(G, 2))
    idx_lin = idx_t3.reshape(N * KNN)
    era7, pan7, cer2 = _sc_gather(era3, pan3, cera2, idx_lin)
    def detile(x):
        x = x.reshape(B, C, KNN, 6, 8, 8, 128)
        x = x.transpose(0, 1, 4, 6, 2, 3, 5)
        return x.reshape(B, C, N, KNN, L)
    era_k = detile(era7)
    pan_k = detile(pan7)
    cera_k = cer2.transpose(2, 0, 1)
    return era_k, pan_k, cera_k
